# baseline (speedup 1.0000x reference)
import jax
import jax.numpy as jnp
from jax import lax
from jax.experimental import pallas as pl
from jax.experimental.pallas import tpu as pltpu
from jax.experimental.pallas import tpu_sc as plsc

BATCH = 16384
EMB = 64
NROWS = 1000000
_NC, _NS = 2, 16
_NW = _NC * _NS
_CB = 256
_RW = 31232
_NBLK = _RW // _CB
_PAD_END = 1000064
_LCAP = 2048 + 16
_BCAP = 256 + 16
_OUTROWS = BATCH + 1024


def _sc_body(user_h, item_h, t0, t1, t2, t3, outU, outI,
             idx_all, bufs0, bufs1, stage, stage_b,
             hrU, hbU, hrI, hbI, brs, bbs,
             semA0, semA1, semB0, semB1, semS):
    wid = lax.axis_index("s") * _NC + lax.axis_index("c")
    lo = wid * _RW
    is_last = wid == _NW - 1
    hi = jnp.where(is_last, NROWS, lo + _RW)
    nblk = jnp.where(is_last, _NBLK + 2, _NBLK)
    iota16 = lax.iota(jnp.int32, 16)
    dump0 = jnp.int32(BATCH)

    def scan(idx_h, hr, hb):
        pltpu.sync_copy(idx_h, idx_all)

        def chunk(c, cnt):
            for k in range(8):
                v = idx_all[pl.ds(c * 128 + k * 16, 16)]
                m = (v >= lo) & (v < hi)
                n = plsc.all_reduce_population_count(m)[0]
                plsc.store_compressed(hr.at[pl.ds(cnt, 16)], v - lo, mask=m)
                b = jnp.full((16,), c * 128 + k * 16, jnp.int32) + iota16
                plsc.store_compressed(hb.at[pl.ds(cnt, 16)], b, mask=m)
                cnt = cnt + n
            return cnt

        return lax.fori_loop(0, BATCH // 128, chunk, jnp.int32(0))

    cntU = scan(user_h, hrU, hbU)
    cntI = scan(item_h, hrI, hbI)

    def reset_stage_b():
        for k in range(8):
            stage_b[pl.ds(k * 16, 16)] = dump0 + k * 16 + iota16

    reset_stage_b()

    def run_pair(ta, tb, out, hr, hb, cnt, semA, semB):
        ngrp = (cnt + 15) // 16

        def fire(b, par):
            c0 = lo + b * _CB
            bufa, bufb = bufs0.at[par], bufs1.at[par]
            pltpu.async_copy(ta.at[:, pl.ds(c0, _CB)], bufa, semA[par])
            pltpu.async_copy(tb.at[:, pl.ds(c0, _CB)], bufb, semB[par])

        def extract(blk, bufa, bufb, sp):
            def coll(g, bn):
                rv = hr[pl.ds(g * 16, 16)]
                bv = hb[pl.ds(g * 16, 16)]
                m = (lax.shift_right_logical(rv, 8) == blk) & (iota16 < cnt - g * 16)
                n = plsc.all_reduce_population_count(m)[0]
                plsc.store_compressed(brs.at[pl.ds(bn, 16)], rv & (_CB - 1), mask=m)
                plsc.store_compressed(bbs.at[pl.ds(bn, 16)], bv, mask=m)
                return bn + n

            bn = lax.fori_loop(0, ngrp, coll, jnp.int32(0))

            def hit_grp(g, sp):
                colsv = brs[pl.ds(g * 16, 16)]
                bvv = bbs[pl.ds(g * 16, 16)]
                for j in range(16):
                    valid = g * 16 + j < bn
                    col = jnp.take(colsv, jnp.full((16,), j, jnp.int32))
                    bsp = jnp.take(bvv, jnp.full((16,), j, jnp.int32))

                    @pl.when(valid)
                    def _(sp=sp, col=col, bsp=bsp):
                        for grp in range(4):
                            va = plsc.load_gather(bufa,
                                                  [iota16 + grp * 16, col])
                            stage[sp, pl.ds(grp * 16, 16)] = va
                            vb = plsc.load_gather(bufb,
                                                  [iota16 + grp * 16, col])
                            stage[sp, pl.ds(64 + grp * 16, 16)] = vb
                        plsc.store_scatter(stage_b,
                                           [jnp.full((16,), sp, jnp.int32)],
                                           bsp, mask=iota16 == 0)

                    sp = sp + valid.astype(jnp.int32)

                    @pl.when(sp >= 112)
                    def _():
                        pltpu.async_copy(stage, out.at[stage_b], semS).wait()
                        reset_stage_b()

                    sp = jnp.where(sp >= 112, 0, sp)
                return sp

            return lax.fori_loop(0, (bn + 15) // 16, hit_grp, sp)

        fire(0, 0)

        def blk2_body(b2, sp):
            for par in range(2):
                b = b2 * 2 + par

                @pl.when(b + 1 < nblk)
                def _(b=b, par=par):
                    fire(b + 1, 1 - par)

                pltpu.make_async_copy(ta.at[:, pl.ds(0, _CB)], bufs0.at[par],
                                      semA[par]).wait()
                pltpu.make_async_copy(tb.at[:, pl.ds(0, _CB)], bufs1.at[par],
                                      semB[par]).wait()
                sp = extract(b, bufs0.at[par], bufs1.at[par], sp)
            return sp

        sp = lax.fori_loop(0, nblk // 2, blk2_body, jnp.int32(0))

        @pl.when(is_last)
        def _():
            c0 = lo + (_NBLK + 2) * _CB
            pltpu.sync_copy(ta.at[:, pl.ds(c0, 128)],
                            bufs0.at[0, :, pl.ds(0, 128)])
            pltpu.sync_copy(tb.at[:, pl.ds(c0, 128)],
                            bufs1.at[0, :, pl.ds(0, 128)])

        sp_t = extract(_NBLK + 2, bufs0.at[0], bufs1.at[0], sp)

        @pl.when(sp_t > 0)
        def _():
            pltpu.async_copy(stage, out.at[stage_b], semS).wait()

        reset_stage_b()

    run_pair(t0, t2, outU, hrU, hbU, cntU, (semA0, semA1), (semB0, semB1))
    run_pair(t1, t3, outI, hrI, hbI, cntI, (semA0, semA1), (semB0, semB1))


def _build_sc():
    return pl.kernel(
        _sc_body,
        out_type=[jax.ShapeDtypeStruct((_OUTROWS, 128), jnp.float32)] * 2,
        mesh=plsc.VectorSubcoreMesh(core_axis_name="c", subcore_axis_name="s",
                                    num_cores=_NC, num_subcores=_NS),
        compiler_params=pltpu.CompilerParams(needs_layout_passes=False),
        scratch_types=[
            pltpu.VMEM((BATCH,), jnp.int32),
            pltpu.VMEM((2, EMB, _CB), jnp.float32),
            pltpu.VMEM((2, EMB, _CB), jnp.float32),
            pltpu.VMEM((128, 128), jnp.float32),
            pltpu.VMEM((128,), jnp.int32),
            pltpu.VMEM((_LCAP,), jnp.int32),
            pltpu.VMEM((_LCAP,), jnp.int32),
            pltpu.VMEM((_LCAP,), jnp.int32),
            pltpu.VMEM((_LCAP,), jnp.int32),
            pltpu.VMEM((_BCAP,), jnp.int32),
            pltpu.VMEM((_BCAP,), jnp.int32),
            pltpu.SemaphoreType.DMA,
            pltpu.SemaphoreType.DMA,
            pltpu.SemaphoreType.DMA,
            pltpu.SemaphoreType.DMA,
            pltpu.SemaphoreType.DMA,
        ],
    )


_BLK = 1024


def _tc_mlp_body(U, I, w1u, w1i, b1, w2, b2, w3, b3, wpg, wph, bp, out):
    u = U[...]
    i = I[...]
    ug = u[:, :EMB]
    um = u[:, EMB:]
    ig = i[:, :EMB]
    im = i[:, EMB:]
    gmf = ug * ig
    h = jnp.dot(um, w1u[...], preferred_element_type=jnp.float32)
    h = h + jnp.dot(im, w1i[...], preferred_element_type=jnp.float32)
    h = jnp.maximum(h + b1[...], 0.0)
    h = jnp.maximum(
        jnp.dot(h, w2[...], preferred_element_type=jnp.float32) + b2[...], 0.0)
    h = jnp.maximum(
        jnp.dot(h, w3[...], preferred_element_type=jnp.float32) + b3[...], 0.0)
    pred = (jnp.sum(gmf * wpg[...], axis=1)
            + jnp.sum(h * wph[...], axis=1) + bp[0, 0])
    out[...] = pred


def _tc_mlp(U, I, w1u, w1i, b1, w2, b2, w3, b3, wpg, wph, bp):
    act_spec = pl.BlockSpec((_BLK, 128), lambda i: (i, 0))
    return pl.pallas_call(
        _tc_mlp_body,
        grid=(BATCH // _BLK,),
        in_specs=[
            act_spec, act_spec,
            pl.BlockSpec((EMB, 128), lambda i: (0, 0)),
            pl.BlockSpec((EMB, 128), lambda i: (0, 0)),
            pl.BlockSpec((1, 128), lambda i: (0, 0)),
            pl.BlockSpec((128, EMB), lambda i: (0, 0)),
            pl.BlockSpec((1, EMB), lambda i: (0, 0)),
            pl.BlockSpec((EMB, 32), lambda i: (0, 0)),
            pl.BlockSpec((1, 32), lambda i: (0, 0)),
            pl.BlockSpec((1, EMB), lambda i: (0, 0)),
            pl.BlockSpec((1, 32), lambda i: (0, 0)),
            pl.BlockSpec((1, 1), lambda i: (0, 0)),
        ],
        out_specs=pl.BlockSpec((_BLK,), lambda i: (i,)),
        out_shape=jax.ShapeDtypeStruct((BATCH,), jnp.float32),
    )(U, I, w1u, w1i, b1, w2, b2, w3, b3, wpg, wph, bp)


def kernel(user, item, eu_gmf, ei_gmf, eu_mlp, ei_mlp,
           W1, b1, W2, b2, W3, b3, Wp, bp):
    user = user.astype(jnp.int32)
    item = item.astype(jnp.int32)
    U, I = _build_sc()(user, item, eu_gmf.T, ei_gmf.T, eu_mlp.T, ei_mlp.T)
    return _tc_mlp(U, I,
                   W1[:, :EMB].T, W1[:, EMB:].T, b1.reshape(1, -1),
                   W2.T, b2.reshape(1, -1), W3.T, b3.reshape(1, -1),
                   Wp[:, :EMB], Wp[:, EMB:], bp.reshape(1, 1))


# speedup vs baseline: 1.0978x; 1.0978x over previous
import jax
import jax.numpy as jnp
from jax import lax
from jax.experimental import pallas as pl
from jax.experimental.pallas import tpu as pltpu
from jax.experimental.pallas import tpu_sc as plsc

BATCH = 16384
EMB = 64
NROWS = 1000000
_NC, _NS = 2, 16
_NW = _NC * _NS
_CB = 256
_RW = 31232
_NBLK = _RW // _CB
_PAD_END = 1000064
_LCAP = 2048 + 16
_BK = 64
_OUTROWS = BATCH + 1024


def _sc_body(user_h, item_h, t0, t1, t2, t3, outU, outI,
             idx_all, bufs0, bufs1, stage, stage_b,
             hrU, hbU, hrI, hbI, bkr, bkb, bcnt,
             semA0, semA1, semB0, semB1, semS):
    wid = lax.axis_index("s") * _NC + lax.axis_index("c")
    lo = wid * _RW
    is_last = wid == _NW - 1
    hi = jnp.where(is_last, NROWS, lo + _RW)
    nblk = jnp.where(is_last, _NBLK + 2, _NBLK)
    iota16 = lax.iota(jnp.int32, 16)
    dump0 = jnp.int32(BATCH)

    def scan(idx_h, hr, hb):
        pltpu.sync_copy(idx_h, idx_all)

        def chunk(c, cnt):
            for k in range(8):
                v = idx_all[pl.ds(c * 128 + k * 16, 16)]
                m = (v >= lo) & (v < hi)
                n = plsc.all_reduce_population_count(m)[0]
                plsc.store_compressed(hr.at[pl.ds(cnt, 16)], v - lo, mask=m)
                b = jnp.full((16,), c * 128 + k * 16, jnp.int32) + iota16
                plsc.store_compressed(hb.at[pl.ds(cnt, 16)], b, mask=m)
                cnt = cnt + n
            return cnt

        return lax.fori_loop(0, BATCH // 128, chunk, jnp.int32(0))

    cntU = scan(user_h, hrU, hbU)
    cntI = scan(item_h, hrI, hbI)

    def reset_stage_b():
        for k in range(8):
            stage_b[pl.ds(k * 16, 16)] = dump0 + k * 16 + iota16

    reset_stage_b()

    def run_pair(ta, tb, out, hr, hb, cnt, semA, semB):
        ngrp = (cnt + 15) // 16

        for k in range(8):
            bcnt[pl.ds(k * 16, 16)] = jnp.zeros((16,), jnp.int32)

        def bucket_grp(g, carry):
            rv = hr[pl.ds(g * 16, 16)]
            bv = hb[pl.ds(g * 16, 16)]
            for j in range(16):
                valid = g * 16 + j < cnt

                @pl.when(valid)
                def _(rv=rv, bv=bv, j=j):
                    r = jnp.take(rv, jnp.full((16,), j, jnp.int32))
                    b = jnp.take(bv, jnp.full((16,), j, jnp.int32))
                    blkv = lax.shift_right_logical(r, 8)
                    n = plsc.load_gather(bcnt, [blkv])
                    n = jnp.minimum(n, _BK - 1)
                    slot = blkv * _BK + n
                    lane0 = iota16 == 0
                    plsc.store_scatter(bkr, [slot], r & (_CB - 1), mask=lane0)
                    plsc.store_scatter(bkb, [slot], b, mask=lane0)
                    plsc.store_scatter(bcnt, [blkv], n + 1, mask=lane0)

            return carry

        lax.fori_loop(0, ngrp, bucket_grp, jnp.int32(0))

        def fire(b, par):
            c0 = lo + b * _CB
            bufa, bufb = bufs0.at[par], bufs1.at[par]
            pltpu.async_copy(ta.at[:, pl.ds(c0, _CB)], bufa, semA[par])
            pltpu.async_copy(tb.at[:, pl.ds(c0, _CB)], bufb, semB[par])

        def extract(blk, bufa, bufb, sp):
            bn = jnp.take(plsc.load_gather(bcnt, [jnp.full((16,), blk,
                                                           jnp.int32)]),
                          jnp.full((16,), 0, jnp.int32))[0]

            def hit_grp(g, sp):
                colsv = bkr[pl.ds(blk * _BK + g * 16, 16)]
                bvv = bkb[pl.ds(blk * _BK + g * 16, 16)]
                for j in range(16):
                    valid = g * 16 + j < bn
                    col = jnp.take(colsv, jnp.full((16,), j, jnp.int32))
                    bsp = jnp.take(bvv, jnp.full((16,), j, jnp.int32))

                    @pl.when(valid)
                    def _(sp=sp, col=col, bsp=bsp):
                        for grp in range(4):
                            va = plsc.load_gather(bufa,
                                                  [iota16 + grp * 16, col])
                            stage[sp, pl.ds(grp * 16, 16)] = va
                            vb = plsc.load_gather(bufb,
                                                  [iota16 + grp * 16, col])
                            stage[sp, pl.ds(64 + grp * 16, 16)] = vb
                        plsc.store_scatter(stage_b,
                                           [jnp.full((16,), sp, jnp.int32)],
                                           bsp, mask=iota16 == 0)

                    sp = sp + valid.astype(jnp.int32)

                @pl.when(sp >= 112)
                def _():
                    pltpu.async_copy(stage, out.at[stage_b], semS).wait()
                    reset_stage_b()

                return jnp.where(sp >= 112, 0, sp)

            return lax.fori_loop(0, (bn + 15) // 16, hit_grp, sp)

        fire(0, 0)

        def blk2_body(b2, sp):
            for par in range(2):
                b = b2 * 2 + par

                @pl.when(b + 1 < nblk)
                def _(b=b, par=par):
                    fire(b + 1, 1 - par)

                pltpu.make_async_copy(ta.at[:, pl.ds(0, _CB)], bufs0.at[par],
                                      semA[par]).wait()
                pltpu.make_async_copy(tb.at[:, pl.ds(0, _CB)], bufs1.at[par],
                                      semB[par]).wait()
                sp = extract(b, bufs0.at[par], bufs1.at[par], sp)
            return sp

        sp = lax.fori_loop(0, nblk // 2, blk2_body, jnp.int32(0))

        @pl.when(is_last)
        def _():
            c0 = lo + (_NBLK + 2) * _CB
            pltpu.sync_copy(ta.at[:, pl.ds(c0, 128)],
                            bufs0.at[0, :, pl.ds(0, 128)])
            pltpu.sync_copy(tb.at[:, pl.ds(c0, 128)],
                            bufs1.at[0, :, pl.ds(0, 128)])

        sp_t = extract(_NBLK + 2, bufs0.at[0], bufs1.at[0], sp)

        @pl.when(sp_t > 0)
        def _():
            pltpu.async_copy(stage, out.at[stage_b], semS).wait()

        reset_stage_b()

    run_pair(t0, t2, outU, hrU, hbU, cntU, (semA0, semA1), (semB0, semB1))
    run_pair(t1, t3, outI, hrI, hbI, cntI, (semA0, semA1), (semB0, semB1))


def _build_sc():
    return pl.kernel(
        _sc_body,
        out_type=[jax.ShapeDtypeStruct((_OUTROWS, 128), jnp.float32)] * 2,
        mesh=plsc.VectorSubcoreMesh(core_axis_name="c", subcore_axis_name="s",
                                    num_cores=_NC, num_subcores=_NS),
        compiler_params=pltpu.CompilerParams(needs_layout_passes=False),
        scratch_types=[
            pltpu.VMEM((BATCH,), jnp.int32),
            pltpu.VMEM((2, EMB, _CB), jnp.float32),
            pltpu.VMEM((2, EMB, _CB), jnp.float32),
            pltpu.VMEM((128, 128), jnp.float32),
            pltpu.VMEM((128,), jnp.int32),
            pltpu.VMEM((_LCAP,), jnp.int32),
            pltpu.VMEM((_LCAP,), jnp.int32),
            pltpu.VMEM((_LCAP,), jnp.int32),
            pltpu.VMEM((_LCAP,), jnp.int32),
            pltpu.VMEM((8192,), jnp.int32),
            pltpu.VMEM((8192,), jnp.int32),
            pltpu.VMEM((128,), jnp.int32),
            pltpu.SemaphoreType.DMA,
            pltpu.SemaphoreType.DMA,
            pltpu.SemaphoreType.DMA,
            pltpu.SemaphoreType.DMA,
            pltpu.SemaphoreType.DMA,
        ],
    )


_BLK = 1024


def _tc_mlp_body(U, I, w1u, w1i, b1, w2, b2, w3, b3, wpg, wph, bp, out):
    u = U[...]
    i = I[...]
    ug = u[:, :EMB]
    um = u[:, EMB:]
    ig = i[:, :EMB]
    im = i[:, EMB:]
    gmf = ug * ig
    h = jnp.dot(um, w1u[...], preferred_element_type=jnp.float32)
    h = h + jnp.dot(im, w1i[...], preferred_element_type=jnp.float32)
    h = jnp.maximum(h + b1[...], 0.0)
    h = jnp.maximum(
        jnp.dot(h, w2[...], preferred_element_type=jnp.float32) + b2[...], 0.0)
    h = jnp.maximum(
        jnp.dot(h, w3[...], preferred_element_type=jnp.float32) + b3[...], 0.0)
    pred = (jnp.sum(gmf * wpg[...], axis=1)
            + jnp.sum(h * wph[...], axis=1) + bp[0, 0])
    out[...] = pred


def _tc_mlp(U, I, w1u, w1i, b1, w2, b2, w3, b3, wpg, wph, bp):
    act_spec = pl.BlockSpec((_BLK, 128), lambda i: (i, 0))
    return pl.pallas_call(
        _tc_mlp_body,
        grid=(BATCH // _BLK,),
        in_specs=[
            act_spec, act_spec,
            pl.BlockSpec((EMB, 128), lambda i: (0, 0)),
            pl.BlockSpec((EMB, 128), lambda i: (0, 0)),
            pl.BlockSpec((1, 128), lambda i: (0, 0)),
            pl.BlockSpec((128, EMB), lambda i: (0, 0)),
            pl.BlockSpec((1, EMB), lambda i: (0, 0)),
            pl.BlockSpec((EMB, 32), lambda i: (0, 0)),
            pl.BlockSpec((1, 32), lambda i: (0, 0)),
            pl.BlockSpec((1, EMB), lambda i: (0, 0)),
            pl.BlockSpec((1, 32), lambda i: (0, 0)),
            pl.BlockSpec((1, 1), lambda i: (0, 0)),
        ],
        out_specs=pl.BlockSpec((_BLK,), lambda i: (i,)),
        out_shape=jax.ShapeDtypeStruct((BATCH,), jnp.float32),
    )(U, I, w1u, w1i, b1, w2, b2, w3, b3, wpg, wph, bp)


def kernel(user, item, eu_gmf, ei_gmf, eu_mlp, ei_mlp,
           W1, b1, W2, b2, W3, b3, Wp, bp):
    user = user.astype(jnp.int32)
    item = item.astype(jnp.int32)
    U, I = _build_sc()(user, item, eu_gmf.T, ei_gmf.T, eu_mlp.T, ei_mlp.T)
    return _tc_mlp(U, I,
                   W1[:, :EMB].T, W1[:, EMB:].T, b1.reshape(1, -1),
                   W2.T, b2.reshape(1, -1), W3.T, b3.reshape(1, -1),
                   Wp[:, :EMB], Wp[:, EMB:], bp.reshape(1, 1))


# speedup vs baseline: 1.1120x; 1.0130x over previous
import jax
import jax.numpy as jnp
from jax import lax
from jax.experimental import pallas as pl
from jax.experimental.pallas import tpu as pltpu
from jax.experimental.pallas import tpu_sc as plsc

BATCH = 16384
EMB = 64
NROWS = 1000000
_NC, _NS = 2, 16
_NW = _NC * _NS
_CB = 256
_RW = 31232
_NBLK = _RW // _CB
_PAD_END = 1000064
_LCAP = 2048 + 16
_BK = 64
_OUTROWS = BATCH + 1024


def _sc_body(user_h, item_h, t0, t1, t2, t3, outU, outI,
             idx_all, bufs0, bufs1, stage, stage_b,
             hrU, hbU, hrI, hbI, bkr, bkb, bcnt,
             semA0, semA1, semB0, semB1, semS):
    wid = lax.axis_index("s") * _NC + lax.axis_index("c")
    lo = wid * _RW
    is_last = wid == _NW - 1
    hi = jnp.where(is_last, NROWS, lo + _RW)
    nblk = jnp.where(is_last, _NBLK + 2, _NBLK)
    iota16 = lax.iota(jnp.int32, 16)
    dump0 = jnp.int32(BATCH)

    def scan(idx_h, hr, hb):
        pltpu.sync_copy(idx_h, idx_all)

        def chunk(c, cnt):
            for k in range(8):
                v = idx_all[pl.ds(c * 128 + k * 16, 16)]
                m = (v >= lo) & (v < hi)
                n = plsc.all_reduce_population_count(m)[0]
                plsc.store_compressed(hr.at[pl.ds(cnt, 16)], v - lo, mask=m)
                b = jnp.full((16,), c * 128 + k * 16, jnp.int32) + iota16
                plsc.store_compressed(hb.at[pl.ds(cnt, 16)], b, mask=m)
                cnt = cnt + n
            return cnt

        return lax.fori_loop(0, BATCH // 128, chunk, jnp.int32(0))

    cntU = scan(user_h, hrU, hbU)
    cntI = scan(item_h, hrI, hbI)

    def reset_stage_b():
        for k in range(8):
            stage_b[pl.ds(k * 16, 16)] = dump0 + k * 16 + iota16

    reset_stage_b()

    def run_pair(ta, tb, out, hr, hb, cnt, semA, semB):
        ngrp = (cnt + 15) // 16

        def fire(b, par):
            c0 = lo + b * _CB
            bufa, bufb = bufs0.at[par], bufs1.at[par]
            pltpu.async_copy(ta.at[:, pl.ds(c0, _CB)], bufa, semA[par])
            pltpu.async_copy(tb.at[:, pl.ds(c0, _CB)], bufb, semB[par])

        fire(0, 0)
        fire(1, 1)

        for k in range(8):
            bcnt[pl.ds(k * 16, 16)] = jnp.zeros((16,), jnp.int32)

        def bucket_grp(g, carry):
            rv = hr[pl.ds(g * 16, 16)]
            bv = hb[pl.ds(g * 16, 16)]
            for j in range(16):
                valid = g * 16 + j < cnt

                @pl.when(valid)
                def _(rv=rv, bv=bv, j=j):
                    r = jnp.take(rv, jnp.full((16,), j, jnp.int32))
                    b = jnp.take(bv, jnp.full((16,), j, jnp.int32))
                    blkv = lax.shift_right_logical(r, 8)
                    n = plsc.load_gather(bcnt, [blkv])
                    n = jnp.minimum(n, _BK - 1)
                    slot = blkv * _BK + n
                    lane0 = iota16 == 0
                    plsc.store_scatter(bkr, [slot], r & (_CB - 1), mask=lane0)
                    plsc.store_scatter(bkb, [slot], b, mask=lane0)
                    plsc.store_scatter(bcnt, [blkv], n + 1, mask=lane0)

            return carry

        lax.fori_loop(0, ngrp, bucket_grp, jnp.int32(0))

        def extract(blk, bufa, bufb, sp):
            bn = jnp.take(plsc.load_gather(bcnt, [jnp.full((16,), blk,
                                                           jnp.int32)]),
                          jnp.full((16,), 0, jnp.int32))[0]

            def hit_grp(g, sp):
                colsv = bkr[pl.ds(blk * _BK + g * 16, 16)]
                bvv = bkb[pl.ds(blk * _BK + g * 16, 16)]
                for j in range(16):
                    valid = g * 16 + j < bn
                    col = jnp.take(colsv, jnp.full((16,), j, jnp.int32))
                    bsp = jnp.take(bvv, jnp.full((16,), j, jnp.int32))

                    @pl.when(valid)
                    def _(sp=sp, col=col, bsp=bsp):
                        for grp in range(4):
                            va = plsc.load_gather(bufa,
                                                  [iota16 + grp * 16, col])
                            stage[sp, pl.ds(grp * 16, 16)] = va
                            vb = plsc.load_gather(bufb,
                                                  [iota16 + grp * 16, col])
                            stage[sp, pl.ds(64 + grp * 16, 16)] = vb
                        plsc.store_scatter(stage_b,
                                           [jnp.full((16,), sp, jnp.int32)],
                                           bsp, mask=iota16 == 0)

                    sp = sp + valid.astype(jnp.int32)

                @pl.when(sp >= 112)
                def _():
                    pltpu.async_copy(stage, out.at[stage_b], semS).wait()
                    reset_stage_b()

                return jnp.where(sp >= 112, 0, sp)

            return lax.fori_loop(0, (bn + 15) // 16, hit_grp, sp)

        def blk2_body(b2, sp):
            for par in range(2):
                b = b2 * 2 + par
                pltpu.make_async_copy(ta.at[:, pl.ds(0, _CB)], bufs0.at[par],
                                      semA[par]).wait()
                pltpu.make_async_copy(tb.at[:, pl.ds(0, _CB)], bufs1.at[par],
                                      semB[par]).wait()
                sp = extract(b, bufs0.at[par], bufs1.at[par], sp)

                @pl.when(b + 2 < nblk)
                def _(b=b, par=par):
                    fire(b + 2, par)

            return sp

        sp = lax.fori_loop(0, nblk // 2, blk2_body, jnp.int32(0))

        @pl.when(is_last)
        def _():
            c0 = lo + (_NBLK + 2) * _CB
            pltpu.sync_copy(ta.at[:, pl.ds(c0, 128)],
                            bufs0.at[0, :, pl.ds(0, 128)])
            pltpu.sync_copy(tb.at[:, pl.ds(c0, 128)],
                            bufs1.at[0, :, pl.ds(0, 128)])

        sp_t = extract(_NBLK + 2, bufs0.at[0], bufs1.at[0], sp)

        @pl.when(sp_t > 0)
        def _():
            pltpu.async_copy(stage, out.at[stage_b], semS).wait()

        reset_stage_b()

    run_pair(t0, t2, outU, hrU, hbU, cntU, (semA0, semA1), (semB0, semB1))
    run_pair(t1, t3, outI, hrI, hbI, cntI, (semA0, semA1), (semB0, semB1))


def _build_sc():
    return pl.kernel(
        _sc_body,
        out_type=[jax.ShapeDtypeStruct((_OUTROWS, 128), jnp.float32)] * 2,
        mesh=plsc.VectorSubcoreMesh(core_axis_name="c", subcore_axis_name="s",
                                    num_cores=_NC, num_subcores=_NS),
        compiler_params=pltpu.CompilerParams(needs_layout_passes=False),
        scratch_types=[
            pltpu.VMEM((BATCH,), jnp.int32),
            pltpu.VMEM((2, EMB, _CB), jnp.float32),
            pltpu.VMEM((2, EMB, _CB), jnp.float32),
            pltpu.VMEM((128, 128), jnp.float32),
            pltpu.VMEM((128,), jnp.int32),
            pltpu.VMEM((_LCAP,), jnp.int32),
            pltpu.VMEM((_LCAP,), jnp.int32),
            pltpu.VMEM((_LCAP,), jnp.int32),
            pltpu.VMEM((_LCAP,), jnp.int32),
            pltpu.VMEM((8192,), jnp.int32),
            pltpu.VMEM((8192,), jnp.int32),
            pltpu.VMEM((128,), jnp.int32),
            pltpu.SemaphoreType.DMA,
            pltpu.SemaphoreType.DMA,
            pltpu.SemaphoreType.DMA,
            pltpu.SemaphoreType.DMA,
            pltpu.SemaphoreType.DMA,
        ],
    )


_BLK = 1024


def _tc_mlp_body(U, I, w1u, w1i, b1, w2, b2, w3, b3, wpg, wph, bp, out):
    u = U[...]
    i = I[...]
    ug = u[:, :EMB]
    um = u[:, EMB:]
    ig = i[:, :EMB]
    im = i[:, EMB:]
    gmf = ug * ig
    h = jnp.dot(um, w1u[...], preferred_element_type=jnp.float32)
    h = h + jnp.dot(im, w1i[...], preferred_element_type=jnp.float32)
    h = jnp.maximum(h + b1[...], 0.0)
    h = jnp.maximum(
        jnp.dot(h, w2[...], preferred_element_type=jnp.float32) + b2[...], 0.0)
    h = jnp.maximum(
        jnp.dot(h, w3[...], preferred_element_type=jnp.float32) + b3[...], 0.0)
    pred = (jnp.sum(gmf * wpg[...], axis=1)
            + jnp.sum(h * wph[...], axis=1) + bp[0, 0])
    out[...] = pred


def _tc_mlp(U, I, w1u, w1i, b1, w2, b2, w3, b3, wpg, wph, bp):
    act_spec = pl.BlockSpec((_BLK, 128), lambda i: (i, 0))
    return pl.pallas_call(
        _tc_mlp_body,
        grid=(BATCH // _BLK,),
        in_specs=[
            act_spec, act_spec,
            pl.BlockSpec((EMB, 128), lambda i: (0, 0)),
            pl.BlockSpec((EMB, 128), lambda i: (0, 0)),
            pl.BlockSpec((1, 128), lambda i: (0, 0)),
            pl.BlockSpec((128, EMB), lambda i: (0, 0)),
            pl.BlockSpec((1, EMB), lambda i: (0, 0)),
            pl.BlockSpec((EMB, 32), lambda i: (0, 0)),
            pl.BlockSpec((1, 32), lambda i: (0, 0)),
            pl.BlockSpec((1, EMB), lambda i: (0, 0)),
            pl.BlockSpec((1, 32), lambda i: (0, 0)),
            pl.BlockSpec((1, 1), lambda i: (0, 0)),
        ],
        out_specs=pl.BlockSpec((_BLK,), lambda i: (i,)),
        out_shape=jax.ShapeDtypeStruct((BATCH,), jnp.float32),
    )(U, I, w1u, w1i, b1, w2, b2, w3, b3, wpg, wph, bp)


def kernel(user, item, eu_gmf, ei_gmf, eu_mlp, ei_mlp,
           W1, b1, W2, b2, W3, b3, Wp, bp):
    user = user.astype(jnp.int32)
    item = item.astype(jnp.int32)
    U, I = _build_sc()(user, item, eu_gmf.T, ei_gmf.T, eu_mlp.T, ei_mlp.T)
    return _tc_mlp(U, I,
                   W1[:, :EMB].T, W1[:, EMB:].T, b1.reshape(1, -1),
                   W2.T, b2.reshape(1, -1), W3.T, b3.reshape(1, -1),
                   Wp[:, :EMB], Wp[:, EMB:], bp.reshape(1, 1))
